# baseline (speedup 1.0000x reference)
import functools

import jax
import jax.numpy as jnp
from jax import lax
from jax.experimental import pallas as pl
from jax.experimental.pallas import tpu as pltpu
from jax.experimental.pallas import tpu_sc as plsc

N_NODES = 20000
N_EDGES = 80000
N_GRAPHS = 64

NC = 2
NS = 16
NW = NC * NS
E_PAD = 81920
EPW = E_PAD // NW
CHUNK = 128
NCHUNK = EPW // CHUNK
N_PAD = 20096
ROWS_PER_SUB = N_PAD // NS
BE = 4096


def _sc_mesh():
    return plsc.VectorSubcoreMesh(core_axis_name="c", subcore_axis_name="s")


def _make_gather(inc):
    @functools.partial(
        pl.kernel,
        mesh=_sc_mesh(),
        compiler_params=pltpu.CompilerParams(use_tc_tiling_on_sc=False),
        out_type=jax.ShapeDtypeStruct((E_PAD, inc), jnp.float32),
        scratch_types=[
            pltpu.VMEM((NCHUNK, CHUNK), jnp.int32),
            pltpu.VMEM((EPW, inc), jnp.float32),
            pltpu.SemaphoreType.DMA,
        ],
    )
    def gather(table, idx3d, out, idx_v, rows_v, sem):
        c = lax.axis_index("c")
        s = lax.axis_index("s")
        wid = s * NC + c
        pltpu.sync_copy(idx3d.at[wid], idx_v)
        copies = [
            pltpu.async_copy(
                table.at[idx_v.at[j]], rows_v.at[pl.ds(j * CHUNK, CHUNK)], sem
            )
            for j in range(NCHUNK)
        ]
        for cp in copies:
            cp.wait()
        pltpu.sync_copy(rows_v, out.at[pl.ds(wid * EPW, EPW)])

    return gather


def _make_scatter(outc):
    @functools.partial(
        pl.kernel,
        mesh=_sc_mesh(),
        compiler_params=pltpu.CompilerParams(use_tc_tiling_on_sc=False),
        out_type=jax.ShapeDtypeStruct((NC, N_PAD, outc), jnp.float32),
        scratch_types=[
            pltpu.VMEM((NCHUNK, CHUNK), jnp.int32),
            pltpu.VMEM((EPW, outc), jnp.float32),
            pltpu.VMEM_SHARED((N_PAD, outc), jnp.float32),
            pltpu.SemaphoreType.DMA,
        ],
    )
    def scatter(msg, idx3d, zeros, out, idx_v, msg_v, acc, sem):
        c = lax.axis_index("c")
        s = lax.axis_index("s")
        wid = s * NC + c
        row0 = s * ROWS_PER_SUB
        pltpu.sync_copy(zeros.at[pl.ds(row0, ROWS_PER_SUB)],
                        msg_v.at[pl.ds(0, ROWS_PER_SUB)])
        pltpu.sync_copy(msg_v.at[pl.ds(0, ROWS_PER_SUB)],
                        acc.at[pl.ds(row0, ROWS_PER_SUB)])
        plsc.subcore_barrier()
        pltpu.sync_copy(idx3d.at[wid], idx_v)
        pltpu.sync_copy(msg.at[pl.ds(wid * EPW, EPW)], msg_v)
        for j in range(NCHUNK):
            pltpu.sync_copy(
                msg_v.at[pl.ds(j * CHUNK, CHUNK)], acc.at[idx_v.at[j]], add=True
            )
        plsc.subcore_barrier()
        pltpu.sync_copy(acc.at[pl.ds(row0, ROWS_PER_SUB)],
                        msg_v.at[pl.ds(0, ROWS_PER_SUB)])
        pltpu.sync_copy(msg_v.at[pl.ds(0, ROWS_PER_SUB)],
                        out.at[c, pl.ds(row0, ROWS_PER_SUB)])

    return scatter


def _msg_body(inc, outc, ea_ref, hs_ref, w1_ref, b1_ref, t2_ref, w2p_ref,
              b2r_ref, out_ref):
    hid = jnp.dot(ea_ref[...], w1_ref[...], preferred_element_type=jnp.float32)
    hid = jnp.maximum(hid + b1_ref[...], 0.0)
    th = jnp.tile(hid, (1, inc))
    xr = jnp.dot(hs_ref[...], t2_ref[...],
                 preferred_element_type=jnp.float32)
    msg = jnp.dot(th * xr, w2p_ref[...], preferred_element_type=jnp.float32)
    msg = msg + jnp.dot(hs_ref[...], b2r_ref[...],
                        preferred_element_type=jnp.float32)
    rid = lax.broadcasted_iota(jnp.int32, (BE, 1), 0) + pl.program_id(0) * BE
    out_ref[...] = jnp.where(rid < N_EDGES, msg, 0.0)


def _msg_call(inc, outc, ea, hs, W1, b1, W2, b2):
    de = ea.shape[1]
    ki = 64 * inc
    t2 = jnp.kron(jnp.eye(inc, dtype=jnp.float32),
                  jnp.ones((1, 64), jnp.float32))
    w2p = W2.reshape(64, inc, outc).transpose(1, 0, 2).reshape(ki, outc)
    b2r = b2.reshape(inc, outc)
    body = functools.partial(_msg_body, inc, outc)
    return pl.pallas_call(
        body,
        grid=(E_PAD // BE,),
        in_specs=[
            pl.BlockSpec((BE, de), lambda i: (i, 0)),
            pl.BlockSpec((BE, inc), lambda i: (i, 0)),
            pl.BlockSpec((de, 64), lambda i: (0, 0)),
            pl.BlockSpec((1, 64), lambda i: (0, 0)),
            pl.BlockSpec((inc, ki), lambda i: (0, 0)),
            pl.BlockSpec((ki, outc), lambda i: (0, 0)),
            pl.BlockSpec((inc, outc), lambda i: (0, 0)),
        ],
        out_specs=pl.BlockSpec((BE, outc), lambda i: (i, 0)),
        out_shape=jax.ShapeDtypeStruct((E_PAD, outc), jnp.float32),
    )(ea, hs, W1, b1.reshape(1, 64), t2, w2p, b2r)


BR = 2000
NB = N_NODES // BR


def _stats_body(p0_ref, p1_ref, x_ref, root_ref, bias_ref, val_ref,
                stats_ref):
    val = p0_ref[...] + p1_ref[...]
    val = val + jnp.dot(x_ref[...], root_ref[...],
                        preferred_element_type=jnp.float32)
    val = val + bias_ref[...]
    val_ref[...] = val
    s = jnp.sum(val, axis=0, keepdims=True)
    s2 = jnp.sum(val * val, axis=0, keepdims=True)
    stats_ref[0, :, :] = jnp.concatenate([s, s2], axis=1)


def _norm_body(has_prev, outc, val_ref, st_ref, g_ref, b_ref, *rest):
    if has_prev:
        prev_ref, out_ref = rest
    else:
        (out_ref,) = rest
    tot = jnp.sum(st_ref[...], axis=0, keepdims=True)
    m = tot[:, :outc] * (1.0 / N_NODES)
    ex2 = tot[:, outc:] * (1.0 / N_NODES)
    v = ex2 - m * m
    hn = (val_ref[...] - m) * lax.rsqrt(v + 1e-5) * g_ref[...] + b_ref[...]
    hn = jnp.maximum(hn, 0.0)
    if has_prev:
        hn = hn + prev_ref[...]
    out_ref[...] = hn


def _post_call(outc, p0, p1, xin, root, bias, gamma, beta, prev=None):
    inc = xin.shape[1]
    val, stats = pl.pallas_call(
        _stats_body,
        grid=(NB,),
        in_specs=[
            pl.BlockSpec((BR, outc), lambda i: (i, 0)),
            pl.BlockSpec((BR, outc), lambda i: (i, 0)),
            pl.BlockSpec((BR, inc), lambda i: (i, 0)),
            pl.BlockSpec((inc, outc), lambda i: (0, 0)),
            pl.BlockSpec((1, outc), lambda i: (0, 0)),
        ],
        out_specs=[
            pl.BlockSpec((BR, outc), lambda i: (i, 0)),
            pl.BlockSpec((1, 1, 2 * outc), lambda i: (i, 0, 0)),
        ],
        out_shape=[
            jax.ShapeDtypeStruct((N_NODES, outc), jnp.float32),
            jax.ShapeDtypeStruct((NB, 1, 2 * outc), jnp.float32),
        ],
    )(p0, p1, xin, root, bias.reshape(1, outc))

    args = [val, stats.reshape(NB, 2 * outc), gamma.reshape(1, outc),
            beta.reshape(1, outc)]
    in_specs = [
        pl.BlockSpec((BR, outc), lambda i: (i, 0)),
        pl.BlockSpec((NB, 2 * outc), lambda i: (0, 0)),
        pl.BlockSpec((1, outc), lambda i: (0, 0)),
        pl.BlockSpec((1, outc), lambda i: (0, 0)),
    ]
    if prev is not None:
        args.append(prev)
        in_specs.append(pl.BlockSpec((BR, outc), lambda i: (i, 0)))
    body = functools.partial(_norm_body, prev is not None, outc)
    return pl.pallas_call(
        body,
        grid=(NB,),
        in_specs=in_specs,
        out_specs=pl.BlockSpec((BR, outc), lambda i: (i, 0)),
        out_shape=jax.ShapeDtypeStruct((N_NODES, outc), jnp.float32),
    )(*args)


def _pool_body(d, h_ref, br_ref, hp_ref, bp_ref, fc1w_ref, fc1b_ref, fc2w_ref,
               fc2b_ref, o0w_ref, o0b_ref, o1w_ref, o1b_ref, out0_ref,
               out1_ref, mx_scr):
    h = h_ref[...]
    br = br_ref[...]
    gid = lax.broadcasted_iota(jnp.int32, (N_GRAPHS, N_NODES), 0)
    oh = (gid == br).astype(jnp.float32)
    s = jnp.dot(oh, h, preferred_element_type=jnp.float32)
    cnt = jnp.sum(oh, axis=1, keepdims=True)
    hp = hp_ref[...]
    bp = bp_ref[...]
    for g in range(N_GRAPHS):
        mx_scr[g, :] = jnp.max(jnp.where(bp == g, hp, -3e38), axis=0)
    mm = mx_scr[...]
    mx = mm[:, :d]
    for j in range(1, 8):
        mx = jnp.maximum(mx, mm[:, j * d:(j + 1) * d])
    mx = jnp.where(cnt > 0, mx, 0.0)
    mean = s / jnp.maximum(cnt, 1.0)
    z = jnp.concatenate([mean, s, mx], axis=1)
    z = jnp.maximum(jnp.dot(z, fc1w_ref[...],
                            preferred_element_type=jnp.float32)
                    + fc1b_ref[...], 0.0)
    z = jnp.maximum(jnp.dot(z, fc2w_ref[...],
                            preferred_element_type=jnp.float32)
                    + fc2b_ref[...], 0.0)
    out0_ref[...] = jnp.dot(z, o0w_ref[...],
                            preferred_element_type=jnp.float32) + o0b_ref[...]
    out1_ref[...] = jnp.dot(z, o1w_ref[...],
                            preferred_element_type=jnp.float32) + o1b_ref[...]


def _pool_call(h, batch, p):
    d = h.shape[1]
    bi = batch.astype(jnp.int32)
    body = functools.partial(_pool_body, d)
    return pl.pallas_call(
        body,
        out_shape=(
            jax.ShapeDtypeStruct((N_GRAPHS, 10), jnp.float32),
            jax.ShapeDtypeStruct((N_GRAPHS, 5), jnp.float32),
        ),
        scratch_shapes=[pltpu.VMEM((N_GRAPHS, 8 * d), jnp.float32)],
    )(
        h,
        bi.reshape(1, N_NODES),
        h.reshape(N_NODES // 8, 8 * d),
        jnp.repeat(bi, d).reshape(N_NODES // 8, 8 * d),
        p['fc1_W'], p['fc1_b'].reshape(1, -1),
        p['fc2_W'], p['fc2_b'].reshape(1, -1),
        p['out0_W'], p['out0_b'].reshape(1, -1),
        p['out1_W'], p['out1_b'].reshape(1, -1),
    )


def kernel(x, edge_index, edge_attr, batch, params):
    p = params
    d = x.shape[1]
    h_dim = p['c1_root'].shape[1]

    src = edge_index[0].astype(jnp.int32)
    dst = edge_index[1].astype(jnp.int32)
    pad = E_PAD - N_EDGES
    src2d = jnp.pad(src, (0, pad)).reshape(NW, NCHUNK, CHUNK)
    dst2d = jnp.pad(dst, (0, pad)).reshape(NW, NCHUNK, CHUNK)
    ea_pad = jnp.pad(edge_attr, ((0, pad), (0, 0)))
    zeros_h = jnp.zeros((N_PAD, h_dim), jnp.float32)
    zeros_d = jnp.zeros((N_PAD, d), jnp.float32)

    gather_d = _make_gather(d)
    gather_h = _make_gather(h_dim)
    scatter_h = _make_scatter(h_dim)
    scatter_d = _make_scatter(d)

    hs = gather_d(x, src2d)
    msg = _msg_call(d, h_dim, ea_pad, hs, p['c1_W1'], p['c1_b1'],
                    p['c1_W2'], p['c1_b2'])
    parts = scatter_h(msg, dst2d, zeros_h)
    h1 = _post_call(h_dim, parts[0, :N_NODES], parts[1, :N_NODES], x,
                    p['c1_root'], p['c1_bias'], p['bn1_g'], p['bn1_b'])

    hs = gather_h(h1, src2d)
    msg = _msg_call(h_dim, h_dim, ea_pad, hs, p['c2_W1'], p['c2_b1'],
                    p['c2_W2'], p['c2_b2'])
    parts = scatter_h(msg, dst2d, zeros_h)
    h2 = _post_call(h_dim, parts[0, :N_NODES], parts[1, :N_NODES], h1,
                    p['c2_root'], p['c2_bias'], p['bn2_g'], p['bn2_b'],
                    prev=h1)

    hs = gather_h(h2, src2d)
    msg = _msg_call(h_dim, d, ea_pad, hs, p['c3_W1'], p['c3_b1'],
                    p['c3_W2'], p['c3_b2'])
    parts = scatter_d(msg, dst2d, zeros_d)
    h3 = _post_call(d, parts[0, :N_NODES], parts[1, :N_NODES], h2,
                    p['c3_root'], p['c3_bias'], p['bn3_g'], p['bn3_b'])

    return _pool_call(h3, batch, p)


# speedup vs baseline: 1.0067x; 1.0067x over previous
import functools

import jax
import jax.numpy as jnp
from jax import lax
from jax.experimental import pallas as pl
from jax.experimental.pallas import tpu as pltpu
from jax.experimental.pallas import tpu_sc as plsc

N_NODES = 20000
N_EDGES = 80000
N_GRAPHS = 64

NC = 2
NS = 16
NW = NC * NS
E_PAD = 81920
EPW = E_PAD // NW
CHUNK = 128
NCHUNK = EPW // CHUNK
N_PAD = 20096
ROWS_PER_SUB = N_PAD // NS
BE = 2048


def _sc_mesh():
    return plsc.VectorSubcoreMesh(core_axis_name="c", subcore_axis_name="s")


def _make_gather(inc):
    @functools.partial(
        pl.kernel,
        mesh=_sc_mesh(),
        compiler_params=pltpu.CompilerParams(use_tc_tiling_on_sc=False),
        out_type=jax.ShapeDtypeStruct((E_PAD, inc), jnp.float32),
        scratch_types=[
            pltpu.VMEM((NCHUNK, CHUNK), jnp.int32),
            pltpu.VMEM((EPW, inc), jnp.float32),
            pltpu.SemaphoreType.DMA,
        ],
    )
    def gather(table, idx3d, out, idx_v, rows_v, sem):
        c = lax.axis_index("c")
        s = lax.axis_index("s")
        wid = s * NC + c
        pltpu.sync_copy(idx3d.at[wid], idx_v)
        copies = [
            pltpu.async_copy(
                table.at[idx_v.at[j]], rows_v.at[pl.ds(j * CHUNK, CHUNK)], sem
            )
            for j in range(NCHUNK)
        ]
        for cp in copies:
            cp.wait()
        pltpu.sync_copy(rows_v, out.at[pl.ds(wid * EPW, EPW)])

    return gather


def _make_scatter(outc):
    @functools.partial(
        pl.kernel,
        mesh=_sc_mesh(),
        compiler_params=pltpu.CompilerParams(use_tc_tiling_on_sc=False),
        out_type=jax.ShapeDtypeStruct((NC, N_PAD, outc), jnp.float32),
        scratch_types=[
            pltpu.VMEM((NCHUNK, CHUNK), jnp.int32),
            pltpu.VMEM((EPW, outc), jnp.float32),
            pltpu.VMEM_SHARED((N_PAD, outc), jnp.float32),
            pltpu.SemaphoreType.DMA,
        ],
    )
    def scatter(msg, idx3d, zeros, out, idx_v, msg_v, acc, sem):
        c = lax.axis_index("c")
        s = lax.axis_index("s")
        wid = s * NC + c
        row0 = s * ROWS_PER_SUB
        za = pltpu.async_copy(zeros.at[pl.ds(row0, ROWS_PER_SUB)],
                              acc.at[pl.ds(row0, ROWS_PER_SUB)], sem)
        ia = pltpu.async_copy(idx3d.at[wid], idx_v, sem)
        ma = pltpu.async_copy(msg.at[pl.ds(wid * EPW, EPW)], msg_v, sem)
        za.wait()
        ia.wait()
        ma.wait()
        plsc.subcore_barrier()
        adds = [
            pltpu.async_copy(msg_v.at[pl.ds(j * CHUNK, CHUNK)],
                             acc.at[idx_v.at[j]], sem, add=True)
            for j in range(NCHUNK)
        ]
        for a in adds:
            a.wait()
        plsc.subcore_barrier()
        pltpu.sync_copy(acc.at[pl.ds(row0, ROWS_PER_SUB)],
                        out.at[c, pl.ds(row0, ROWS_PER_SUB)])

    return scatter


def _msg_body(inc, outc, ea_ref, hs_ref, w1_ref, b1_ref, t2_ref, w2p_ref,
              b2r_ref, out_ref):
    hid = jnp.dot(ea_ref[...], w1_ref[...], preferred_element_type=jnp.float32)
    hid = jnp.maximum(hid + b1_ref[...], 0.0)
    th = jnp.tile(hid, (1, inc))
    xr = jnp.dot(hs_ref[...], t2_ref[...],
                 preferred_element_type=jnp.float32)
    msg = jnp.dot(th * xr, w2p_ref[...], preferred_element_type=jnp.float32)
    msg = msg + jnp.dot(hs_ref[...], b2r_ref[...],
                        preferred_element_type=jnp.float32)
    rid = lax.broadcasted_iota(jnp.int32, (BE, 1), 0) + pl.program_id(0) * BE
    out_ref[...] = jnp.where(rid < N_EDGES, msg, 0.0)


def _msg_call(inc, outc, ea, hs, W1, b1, W2, b2):
    de = ea.shape[1]
    ki = 64 * inc
    t2 = jnp.kron(jnp.eye(inc, dtype=jnp.float32),
                  jnp.ones((1, 64), jnp.float32))
    w2p = W2.reshape(64, inc, outc).transpose(1, 0, 2).reshape(ki, outc)
    b2r = b2.reshape(inc, outc)
    body = functools.partial(_msg_body, inc, outc)
    return pl.pallas_call(
        body,
        grid=(E_PAD // BE,),
        in_specs=[
            pl.BlockSpec((BE, de), lambda i: (i, 0)),
            pl.BlockSpec((BE, inc), lambda i: (i, 0)),
            pl.BlockSpec((de, 64), lambda i: (0, 0)),
            pl.BlockSpec((1, 64), lambda i: (0, 0)),
            pl.BlockSpec((inc, ki), lambda i: (0, 0)),
            pl.BlockSpec((ki, outc), lambda i: (0, 0)),
            pl.BlockSpec((inc, outc), lambda i: (0, 0)),
        ],
        out_specs=pl.BlockSpec((BE, outc), lambda i: (i, 0)),
        out_shape=jax.ShapeDtypeStruct((E_PAD, outc), jnp.float32),
    )(ea, hs, W1, b1.reshape(1, 64), t2, w2p, b2r)


BR = 2000
NB = N_NODES // BR


def _stats_body(p0_ref, p1_ref, x_ref, root_ref, bias_ref, val_ref,
                stats_ref):
    val = p0_ref[...] + p1_ref[...]
    val = val + jnp.dot(x_ref[...], root_ref[...],
                        preferred_element_type=jnp.float32)
    val = val + bias_ref[...]
    val_ref[...] = val
    s = jnp.sum(val, axis=0, keepdims=True)
    s2 = jnp.sum(val * val, axis=0, keepdims=True)
    stats_ref[0, :, :] = jnp.concatenate([s, s2], axis=1)


def _norm_body(has_prev, outc, val_ref, st_ref, g_ref, b_ref, *rest):
    if has_prev:
        prev_ref, out_ref = rest
    else:
        (out_ref,) = rest
    tot = jnp.sum(st_ref[...], axis=0, keepdims=True)
    m = tot[:, :outc] * (1.0 / N_NODES)
    ex2 = tot[:, outc:] * (1.0 / N_NODES)
    v = ex2 - m * m
    hn = (val_ref[...] - m) * lax.rsqrt(v + 1e-5) * g_ref[...] + b_ref[...]
    hn = jnp.maximum(hn, 0.0)
    if has_prev:
        hn = hn + prev_ref[...]
    out_ref[...] = hn


def _post_call(outc, p0, p1, xin, root, bias, gamma, beta, prev=None):
    inc = xin.shape[1]
    val, stats = pl.pallas_call(
        _stats_body,
        grid=(NB,),
        in_specs=[
            pl.BlockSpec((BR, outc), lambda i: (i, 0)),
            pl.BlockSpec((BR, outc), lambda i: (i, 0)),
            pl.BlockSpec((BR, inc), lambda i: (i, 0)),
            pl.BlockSpec((inc, outc), lambda i: (0, 0)),
            pl.BlockSpec((1, outc), lambda i: (0, 0)),
        ],
        out_specs=[
            pl.BlockSpec((BR, outc), lambda i: (i, 0)),
            pl.BlockSpec((1, 1, 2 * outc), lambda i: (i, 0, 0)),
        ],
        out_shape=[
            jax.ShapeDtypeStruct((N_NODES, outc), jnp.float32),
            jax.ShapeDtypeStruct((NB, 1, 2 * outc), jnp.float32),
        ],
    )(p0, p1, xin, root, bias.reshape(1, outc))

    args = [val, stats.reshape(NB, 2 * outc), gamma.reshape(1, outc),
            beta.reshape(1, outc)]
    in_specs = [
        pl.BlockSpec((BR, outc), lambda i: (i, 0)),
        pl.BlockSpec((NB, 2 * outc), lambda i: (0, 0)),
        pl.BlockSpec((1, outc), lambda i: (0, 0)),
        pl.BlockSpec((1, outc), lambda i: (0, 0)),
    ]
    if prev is not None:
        args.append(prev)
        in_specs.append(pl.BlockSpec((BR, outc), lambda i: (i, 0)))
    body = functools.partial(_norm_body, prev is not None, outc)
    return pl.pallas_call(
        body,
        grid=(NB,),
        in_specs=in_specs,
        out_specs=pl.BlockSpec((BR, outc), lambda i: (i, 0)),
        out_shape=jax.ShapeDtypeStruct((N_NODES, outc), jnp.float32),
    )(*args)


def _pool_body(d, h_ref, br_ref, hp_ref, bp_ref, fc1w_ref, fc1b_ref, fc2w_ref,
               fc2b_ref, o0w_ref, o0b_ref, o1w_ref, o1b_ref, out0_ref,
               out1_ref, mx_scr):
    h = h_ref[...]
    br = br_ref[...]
    gid = lax.broadcasted_iota(jnp.int32, (N_GRAPHS, N_NODES), 0)
    oh = (gid == br).astype(jnp.float32)
    s = jnp.dot(oh, h, preferred_element_type=jnp.float32)
    cnt = jnp.sum(oh, axis=1, keepdims=True)
    hp = hp_ref[...]
    bp = bp_ref[...]
    for g in range(N_GRAPHS):
        mx_scr[g, :] = jnp.max(jnp.where(bp == g, hp, -3e38), axis=0)
    mm = mx_scr[...]
    mx = mm[:, :d]
    for j in range(1, 8):
        mx = jnp.maximum(mx, mm[:, j * d:(j + 1) * d])
    mx = jnp.where(cnt > 0, mx, 0.0)
    mean = s / jnp.maximum(cnt, 1.0)
    z = jnp.concatenate([mean, s, mx], axis=1)
    z = jnp.maximum(jnp.dot(z, fc1w_ref[...],
                            preferred_element_type=jnp.float32)
                    + fc1b_ref[...], 0.0)
    z = jnp.maximum(jnp.dot(z, fc2w_ref[...],
                            preferred_element_type=jnp.float32)
                    + fc2b_ref[...], 0.0)
    out0_ref[...] = jnp.dot(z, o0w_ref[...],
                            preferred_element_type=jnp.float32) + o0b_ref[...]
    out1_ref[...] = jnp.dot(z, o1w_ref[...],
                            preferred_element_type=jnp.float32) + o1b_ref[...]


def _pool_call(h, batch, p):
    d = h.shape[1]
    bi = batch.astype(jnp.int32)
    body = functools.partial(_pool_body, d)
    return pl.pallas_call(
        body,
        out_shape=(
            jax.ShapeDtypeStruct((N_GRAPHS, 10), jnp.float32),
            jax.ShapeDtypeStruct((N_GRAPHS, 5), jnp.float32),
        ),
        scratch_shapes=[pltpu.VMEM((N_GRAPHS, 8 * d), jnp.float32)],
    )(
        h,
        bi.reshape(1, N_NODES),
        h.reshape(N_NODES // 8, 8 * d),
        jnp.repeat(bi, d).reshape(N_NODES // 8, 8 * d),
        p['fc1_W'], p['fc1_b'].reshape(1, -1),
        p['fc2_W'], p['fc2_b'].reshape(1, -1),
        p['out0_W'], p['out0_b'].reshape(1, -1),
        p['out1_W'], p['out1_b'].reshape(1, -1),
    )


def kernel(x, edge_index, edge_attr, batch, params):
    p = params
    d = x.shape[1]
    h_dim = p['c1_root'].shape[1]

    src = edge_index[0].astype(jnp.int32)
    dst = edge_index[1].astype(jnp.int32)
    pad = E_PAD - N_EDGES
    src2d = jnp.pad(src, (0, pad)).reshape(NW, NCHUNK, CHUNK)
    dst2d = jnp.pad(dst, (0, pad)).reshape(NW, NCHUNK, CHUNK)
    ea_pad = jnp.pad(edge_attr, ((0, pad), (0, 0)))
    zeros_h = jnp.zeros((N_PAD, h_dim), jnp.float32)
    zeros_d = jnp.zeros((N_PAD, d), jnp.float32)

    gather_d = _make_gather(d)
    gather_h = _make_gather(h_dim)
    scatter_h = _make_scatter(h_dim)
    scatter_d = _make_scatter(d)

    hs = gather_d(x, src2d)
    msg = _msg_call(d, h_dim, ea_pad, hs, p['c1_W1'], p['c1_b1'],
                    p['c1_W2'], p['c1_b2'])
    parts = scatter_h(msg, dst2d, zeros_h)
    h1 = _post_call(h_dim, parts[0, :N_NODES], parts[1, :N_NODES], x,
                    p['c1_root'], p['c1_bias'], p['bn1_g'], p['bn1_b'])

    hs = gather_h(h1, src2d)
    msg = _msg_call(h_dim, h_dim, ea_pad, hs, p['c2_W1'], p['c2_b1'],
                    p['c2_W2'], p['c2_b2'])
    parts = scatter_h(msg, dst2d, zeros_h)
    h2 = _post_call(h_dim, parts[0, :N_NODES], parts[1, :N_NODES], h1,
                    p['c2_root'], p['c2_bias'], p['bn2_g'], p['bn2_b'],
                    prev=h1)

    hs = gather_h(h2, src2d)
    msg = _msg_call(h_dim, d, ea_pad, hs, p['c3_W1'], p['c3_b1'],
                    p['c3_W2'], p['c3_b2'])
    parts = scatter_d(msg, dst2d, zeros_d)
    h3 = _post_call(d, parts[0, :N_NODES], parts[1, :N_NODES], h2,
                    p['c3_root'], p['c3_bias'], p['bn3_g'], p['bn3_b'])

    return _pool_call(h3, batch, p)


# speedup vs baseline: 1.0301x; 1.0233x over previous
import functools

import jax
import jax.numpy as jnp
from jax import lax
from jax.experimental import pallas as pl
from jax.experimental.pallas import tpu as pltpu
from jax.experimental.pallas import tpu_sc as plsc

N_NODES = 20000
N_EDGES = 80000
N_GRAPHS = 64

NC = 2
NS = 16
NW = NC * NS
E_PAD = 81920
EPW = E_PAD // NW
CHUNK = 128
NCHUNK = EPW // CHUNK
N_PAD = 20096
ROWS_PER_SUB = N_PAD // NS
BE = 2048


def _sc_mesh():
    return plsc.VectorSubcoreMesh(core_axis_name="c", subcore_axis_name="s")


def _make_gather(inc):
    @functools.partial(
        pl.kernel,
        mesh=_sc_mesh(),
        compiler_params=pltpu.CompilerParams(use_tc_tiling_on_sc=False),
        out_type=jax.ShapeDtypeStruct((E_PAD, inc), jnp.float32),
        scratch_types=[
            pltpu.VMEM((NCHUNK, CHUNK), jnp.int32),
            pltpu.VMEM((EPW, inc), jnp.float32),
            pltpu.SemaphoreType.DMA,
        ],
    )
    def gather(table, idx3d, out, idx_v, rows_v, sem):
        c = lax.axis_index("c")
        s = lax.axis_index("s")
        wid = s * NC + c
        pltpu.sync_copy(idx3d.at[wid], idx_v)
        copies = [
            pltpu.async_copy(
                table.at[idx_v.at[j]], rows_v.at[pl.ds(j * CHUNK, CHUNK)], sem
            )
            for j in range(NCHUNK)
        ]
        for cp in copies:
            cp.wait()
        pltpu.sync_copy(rows_v, out.at[pl.ds(wid * EPW, EPW)])

    return gather


def _make_scatter(outc):
    @functools.partial(
        pl.kernel,
        mesh=_sc_mesh(),
        compiler_params=pltpu.CompilerParams(use_tc_tiling_on_sc=False),
        out_type=jax.ShapeDtypeStruct((NC, N_PAD, outc), jnp.float32),
        scratch_types=[
            pltpu.VMEM((NCHUNK, CHUNK), jnp.int32),
            pltpu.VMEM((EPW, outc), jnp.float32),
            pltpu.VMEM_SHARED((N_PAD, outc), jnp.float32),
            pltpu.SemaphoreType.DMA,
        ],
    )
    def scatter(msg, idx3d, zeros, out, idx_v, msg_v, acc, sem):
        c = lax.axis_index("c")
        s = lax.axis_index("s")
        wid = s * NC + c
        row0 = s * ROWS_PER_SUB
        za = pltpu.async_copy(zeros.at[pl.ds(row0, ROWS_PER_SUB)],
                              acc.at[pl.ds(row0, ROWS_PER_SUB)], sem)
        ia = pltpu.async_copy(idx3d.at[wid], idx_v, sem)
        ma = pltpu.async_copy(msg.at[pl.ds(wid * EPW, EPW)], msg_v, sem)
        za.wait()
        ia.wait()
        ma.wait()
        plsc.subcore_barrier()
        adds = [
            pltpu.async_copy(msg_v.at[pl.ds(j * CHUNK, CHUNK)],
                             acc.at[idx_v.at[j]], sem, add=True)
            for j in range(NCHUNK)
        ]
        for a in adds:
            a.wait()
        plsc.subcore_barrier()
        pltpu.sync_copy(acc.at[pl.ds(row0, ROWS_PER_SUB)],
                        out.at[c, pl.ds(row0, ROWS_PER_SUB)])

    return scatter


def _msg_body(inc, outc, ea_ref, hs_ref, w1_ref, b1_ref, t2_ref, w2p_ref,
              b2r_ref, out_ref):
    hid = jnp.dot(ea_ref[...], w1_ref[...], preferred_element_type=jnp.float32)
    hid = jnp.maximum(hid + b1_ref[...], 0.0)
    th = jnp.tile(hid, (1, inc))
    xr = jnp.dot(hs_ref[...], t2_ref[...],
                 preferred_element_type=jnp.float32)
    msg = jnp.dot(th * xr, w2p_ref[...], preferred_element_type=jnp.float32)
    msg = msg + jnp.dot(hs_ref[...], b2r_ref[...],
                        preferred_element_type=jnp.float32)
    rid = lax.broadcasted_iota(jnp.int32, (BE, 1), 0) + pl.program_id(0) * BE
    out_ref[...] = jnp.where(rid < N_EDGES, msg, 0.0)


def _msg_call(inc, outc, ea, hs, W1, b1, W2, b2):
    de = ea.shape[1]
    ki = 64 * inc
    t2 = jnp.kron(jnp.eye(inc, dtype=jnp.float32),
                  jnp.ones((1, 64), jnp.float32))
    w2p = W2.reshape(64, inc, outc).transpose(1, 0, 2).reshape(ki, outc)
    b2r = b2.reshape(inc, outc)
    body = functools.partial(_msg_body, inc, outc)
    return pl.pallas_call(
        body,
        grid=(E_PAD // BE,),
        in_specs=[
            pl.BlockSpec((BE, de), lambda i: (i, 0)),
            pl.BlockSpec((BE, inc), lambda i: (i, 0)),
            pl.BlockSpec((de, 64), lambda i: (0, 0)),
            pl.BlockSpec((1, 64), lambda i: (0, 0)),
            pl.BlockSpec((inc, ki), lambda i: (0, 0)),
            pl.BlockSpec((ki, outc), lambda i: (0, 0)),
            pl.BlockSpec((inc, outc), lambda i: (0, 0)),
        ],
        out_specs=pl.BlockSpec((BE, outc), lambda i: (i, 0)),
        out_shape=jax.ShapeDtypeStruct((E_PAD, outc), jnp.float32),
    )(ea, hs, W1, b1.reshape(1, 64), t2, w2p, b2r)


BR = 2000
NB = N_NODES // BR


def _stats_body(p0_ref, p1_ref, x_ref, root_ref, bias_ref, val_ref,
                stats_ref):
    val = p0_ref[...] + p1_ref[...]
    val = val + jnp.dot(x_ref[...], root_ref[...],
                        preferred_element_type=jnp.float32)
    val = val + bias_ref[...]
    val_ref[...] = val
    s = jnp.sum(val, axis=0, keepdims=True)
    s2 = jnp.sum(val * val, axis=0, keepdims=True)
    stats_ref[0, :, :] = jnp.concatenate([s, s2], axis=1)


def _norm_body(has_prev, outc, val_ref, st_ref, g_ref, b_ref, *rest):
    if has_prev:
        prev_ref, out_ref = rest
    else:
        (out_ref,) = rest
    tot = jnp.sum(st_ref[...], axis=0, keepdims=True)
    m = tot[:, :outc] * (1.0 / N_NODES)
    ex2 = tot[:, outc:] * (1.0 / N_NODES)
    v = ex2 - m * m
    hn = (val_ref[...] - m) * lax.rsqrt(v + 1e-5) * g_ref[...] + b_ref[...]
    hn = jnp.maximum(hn, 0.0)
    if has_prev:
        hn = hn + prev_ref[...]
    out_ref[...] = hn


def _post_call(outc, p0, p1, xin, root, bias, gamma, beta, prev=None):
    inc = xin.shape[1]
    val, stats = pl.pallas_call(
        _stats_body,
        grid=(NB,),
        in_specs=[
            pl.BlockSpec((BR, outc), lambda i: (i, 0)),
            pl.BlockSpec((BR, outc), lambda i: (i, 0)),
            pl.BlockSpec((BR, inc), lambda i: (i, 0)),
            pl.BlockSpec((inc, outc), lambda i: (0, 0)),
            pl.BlockSpec((1, outc), lambda i: (0, 0)),
        ],
        out_specs=[
            pl.BlockSpec((BR, outc), lambda i: (i, 0)),
            pl.BlockSpec((1, 1, 2 * outc), lambda i: (i, 0, 0)),
        ],
        out_shape=[
            jax.ShapeDtypeStruct((N_NODES, outc), jnp.float32),
            jax.ShapeDtypeStruct((NB, 1, 2 * outc), jnp.float32),
        ],
    )(p0, p1, xin, root, bias.reshape(1, outc))

    args = [val, stats.reshape(NB, 2 * outc), gamma.reshape(1, outc),
            beta.reshape(1, outc)]
    in_specs = [
        pl.BlockSpec((BR, outc), lambda i: (i, 0)),
        pl.BlockSpec((NB, 2 * outc), lambda i: (0, 0)),
        pl.BlockSpec((1, outc), lambda i: (0, 0)),
        pl.BlockSpec((1, outc), lambda i: (0, 0)),
    ]
    if prev is not None:
        args.append(prev)
        in_specs.append(pl.BlockSpec((BR, outc), lambda i: (i, 0)))
    body = functools.partial(_norm_body, prev is not None, outc)
    return pl.pallas_call(
        body,
        grid=(NB,),
        in_specs=in_specs,
        out_specs=pl.BlockSpec((BR, outc), lambda i: (i, 0)),
        out_shape=jax.ShapeDtypeStruct((N_NODES, outc), jnp.float32),
    )(*args)


def _pool_body(d, h_ref, br_ref, hp_ref, bp_ref, fc1w_ref, fc1b_ref, fc2w_ref,
               fc2b_ref, o0w_ref, o0b_ref, o1w_ref, o1b_ref, out0_ref,
               out1_ref, mx_scr):
    h = h_ref[...]
    br = br_ref[...]
    gid = lax.broadcasted_iota(jnp.int32, (N_GRAPHS, N_NODES), 0)
    oh = (gid == br).astype(jnp.float32)
    s = jnp.dot(oh, h, preferred_element_type=jnp.float32)
    cnt = jnp.sum(oh, axis=1, keepdims=True)
    hp = hp_ref[...]
    bp = bp_ref[...]
    for g in range(N_GRAPHS):
        mx_scr[g, :] = jnp.max(jnp.where(bp == g, hp, -3e38), axis=0)
    mm = mx_scr[...]
    mx = mm[:, :d]
    for j in range(1, 8):
        mx = jnp.maximum(mx, mm[:, j * d:(j + 1) * d])
    mx = jnp.where(cnt > 0, mx, 0.0)
    mean = s / jnp.maximum(cnt, 1.0)
    z = jnp.concatenate([mean, s, mx], axis=1)
    z = jnp.maximum(jnp.dot(z, fc1w_ref[...],
                            preferred_element_type=jnp.float32)
                    + fc1b_ref[...], 0.0)
    z = jnp.maximum(jnp.dot(z, fc2w_ref[...],
                            preferred_element_type=jnp.float32)
                    + fc2b_ref[...], 0.0)
    out0_ref[...] = jnp.dot(z, o0w_ref[...],
                            preferred_element_type=jnp.float32) + o0b_ref[...]
    out1_ref[...] = jnp.dot(z, o1w_ref[...],
                            preferred_element_type=jnp.float32) + o1b_ref[...]


def _pool_call(h, batch, p):
    d = h.shape[1]
    bi = batch.astype(jnp.int32)
    body = functools.partial(_pool_body, d)
    return pl.pallas_call(
        body,
        out_shape=(
            jax.ShapeDtypeStruct((N_GRAPHS, 10), jnp.float32),
            jax.ShapeDtypeStruct((N_GRAPHS, 5), jnp.float32),
        ),
        scratch_shapes=[pltpu.VMEM((N_GRAPHS, 8 * d), jnp.float32)],
    )(
        h,
        bi.reshape(1, N_NODES),
        h.reshape(N_NODES // 8, 8 * d),
        jnp.repeat(bi, d).reshape(N_NODES // 8, 8 * d),
        p['fc1_W'], p['fc1_b'].reshape(1, -1),
        p['fc2_W'], p['fc2_b'].reshape(1, -1),
        p['out0_W'], p['out0_b'].reshape(1, -1),
        p['out1_W'], p['out1_b'].reshape(1, -1),
    )


def kernel(x, edge_index, edge_attr, batch, params):
    p = params
    d = x.shape[1]
    h_dim = p['c1_root'].shape[1]

    src = edge_index[0].astype(jnp.int32)
    dst = edge_index[1].astype(jnp.int32)
    pad = E_PAD - N_EDGES
    src2d = jnp.pad(src, (0, pad)).reshape(NW, NCHUNK, CHUNK)
    dst2d = jnp.pad(dst, (0, pad)).reshape(NW, NCHUNK, CHUNK)
    ea_pad = edge_attr
    zeros_h = jnp.zeros((N_PAD, h_dim), jnp.float32)
    zeros_d = jnp.zeros((N_PAD, d), jnp.float32)

    gather_d = _make_gather(d)
    gather_h = _make_gather(h_dim)
    scatter_h = _make_scatter(h_dim)
    scatter_d = _make_scatter(d)

    hs = gather_d(x, src2d)
    msg = _msg_call(d, h_dim, ea_pad, hs, p['c1_W1'], p['c1_b1'],
                    p['c1_W2'], p['c1_b2'])
    parts = scatter_h(msg, dst2d, zeros_h)
    h1 = _post_call(h_dim, parts[0, :N_NODES], parts[1, :N_NODES], x,
                    p['c1_root'], p['c1_bias'], p['bn1_g'], p['bn1_b'])

    hs = gather_h(h1, src2d)
    msg = _msg_call(h_dim, h_dim, ea_pad, hs, p['c2_W1'], p['c2_b1'],
                    p['c2_W2'], p['c2_b2'])
    parts = scatter_h(msg, dst2d, zeros_h)
    h2 = _post_call(h_dim, parts[0, :N_NODES], parts[1, :N_NODES], h1,
                    p['c2_root'], p['c2_bias'], p['bn2_g'], p['bn2_b'],
                    prev=h1)

    hs = gather_h(h2, src2d)
    msg = _msg_call(h_dim, d, ea_pad, hs, p['c3_W1'], p['c3_b1'],
                    p['c3_W2'], p['c3_b2'])
    parts = scatter_d(msg, dst2d, zeros_d)
    h3 = _post_call(d, parts[0, :N_NODES], parts[1, :N_NODES], h2,
                    p['c3_root'], p['c3_bias'], p['bn3_g'], p['bn3_b'])

    return _pool_call(h3, batch, p)
